# Initial kernel scaffold; baseline (speedup 1.0000x reference)
import jax
import jax.numpy as jnp
from jax.experimental import pallas as pl


def kernel(idx, targets, table):
    raise NotImplementedError("write your pallas kernel here")


# speedup vs baseline: 2.0267x; 2.0267x over previous
import functools

import jax
import jax.numpy as jnp
from jax import lax
from jax.experimental import pallas as pl
from jax.experimental.pallas import tpu as pltpu
from jax.experimental.pallas import tpu_sc as plsc

VOCAB = 1000
N = 51200
NW = 32
PER_W = N // NW
CH = 64
NCH = PER_W // CH
LANES = 16


def _lse_body(t_ref, o_ref):
    x = t_ref[...]
    m = jnp.max(x, axis=1, keepdims=True)
    s = jnp.sum(jnp.exp(x - m), axis=1, keepdims=True)
    o_ref[...] = m + jnp.log(s)


def _fin_body(p_ref, o_ref):
    o_ref[...] = jnp.reshape(jnp.sum(p_ref[...]) * (1.0 / N), (1, 1))


def _sc_gather(idx_hbm, tgt_hbm, table_hbm, lse_hbm, out_hbm, psum_hbm,
               idx_v, tgt_v, lse_v, rows_v, acc_v, gsem):
    wid = lax.axis_index("s") * 2 + lax.axis_index("c")
    base = wid * PER_W

    pltpu.sync_copy(idx_hbm.at[pl.ds(base, PER_W)], idx_v)
    pltpu.sync_copy(tgt_hbm.at[pl.ds(base, PER_W)], tgt_v)
    pltpu.sync_copy(lse_hbm, lse_v)

    iota = lax.iota(jnp.int32, LANES)

    def chunk(c, acc):
        idx_sl = idx_v.at[pl.ds(c * CH, CH)]
        cp = pltpu.make_async_copy(table_hbm.at[idx_sl], rows_v, gsem)
        cp.start()
        cp.wait()
        for g in range(CH // LANES):
            off = c * CH + g * LANES
            tcol = tgt_v[pl.ds(off, LANES)]
            ii = idx_v[pl.ds(off, LANES)]
            rowid = iota + g * LANES
            tval = plsc.load_gather(rows_v, [rowid, tcol])
            lval = plsc.load_gather(lse_v, [ii])
            acc = acc + (lval - tval)
        pltpu.sync_copy(rows_v, out_hbm.at[pl.ds(base + c * CH, CH)])
        return acc

    acc = lax.fori_loop(0, NCH, chunk, jnp.zeros((LANES,), jnp.float32))
    acc_v[...] = acc
    pltpu.sync_copy(acc_v, psum_hbm.at[wid])


def kernel(idx, targets, table):
    idx_flat = idx.reshape(-1).astype(jnp.int32)
    tgt_flat = targets.reshape(-1).astype(jnp.int32)

    lse = pl.pallas_call(
        _lse_body,
        out_shape=jax.ShapeDtypeStruct((VOCAB, 1), jnp.float32),
    )(table)
    lse = lse.reshape(VOCAB)

    sc = functools.partial(
        pl.kernel,
        mesh=plsc.VectorSubcoreMesh(core_axis_name="c", subcore_axis_name="s"),
        out_type=[
            jax.ShapeDtypeStruct((N, VOCAB), jnp.float32),
            jax.ShapeDtypeStruct((NW, LANES), jnp.float32),
        ],
        scratch_types=[
            pltpu.VMEM((PER_W,), jnp.int32),
            pltpu.VMEM((PER_W,), jnp.int32),
            pltpu.VMEM((VOCAB,), jnp.float32),
            pltpu.VMEM((CH, VOCAB), jnp.float32),
            pltpu.VMEM((LANES,), jnp.float32),
            pltpu.SemaphoreType.DMA,
        ],
        compiler_params=pltpu.CompilerParams(
            use_tc_tiling_on_sc=False, needs_layout_passes=False),
    )(_sc_gather)
    logits2, psums = sc(idx_flat, tgt_flat, table, lse)

    fin = pl.pallas_call(
        _fin_body,
        out_shape=jax.ShapeDtypeStruct((1, 1), jnp.float32),
    )(psums)
    loss = fin[0, 0]
    return (logits2, loss)


# speedup vs baseline: 2.0530x; 1.0130x over previous
import functools

import jax
import jax.numpy as jnp
from jax import lax
from jax.experimental import pallas as pl
from jax.experimental.pallas import tpu as pltpu
from jax.experimental.pallas import tpu_sc as plsc

VOCAB = 1000
N = 51200
NW = 32
PER_W = N // NW
CH = 32
NCH = PER_W // CH
NPAIR = NCH // 2
LANES = 16


def _lse_body(t_ref, o_ref):
    x = t_ref[...]
    m = jnp.max(x, axis=1, keepdims=True)
    s = jnp.sum(jnp.exp(x - m), axis=1, keepdims=True)
    o_ref[...] = m + jnp.log(s)


def _fin_body(p_ref, o_ref):
    o_ref[...] = jnp.reshape(jnp.sum(p_ref[...]) * (1.0 / N), (1, 1))


def _sc_gather(idx_hbm, tgt_hbm, table_hbm, lse_hbm, out_hbm, psum_hbm,
               idx_v, tgt_v, lse_v, rows0, rows1, acc_v,
               gsem0, gsem1, osem0, osem1):
    wid = lax.axis_index("s") * 2 + lax.axis_index("c")
    base = wid * PER_W

    pltpu.sync_copy(idx_hbm.at[pl.ds(base, PER_W)], idx_v)
    pltpu.sync_copy(tgt_hbm.at[pl.ds(base, PER_W)], tgt_v)
    pltpu.sync_copy(lse_hbm, lse_v)

    iota = lax.iota(jnp.int32, LANES)

    def gather_start(c, rows, gsem):
        pltpu.make_async_copy(
            table_hbm.at[idx_v.at[pl.ds(c * CH, CH)]], rows, gsem).start()

    def gather_wait(c, rows, gsem):
        pltpu.make_async_copy(
            table_hbm.at[idx_v.at[pl.ds(c * CH, CH)]], rows, gsem).wait()

    def out_start(c, rows, osem):
        pltpu.make_async_copy(
            rows, out_hbm.at[pl.ds(base + c * CH, CH)], osem).start()

    def out_wait(c, rows, osem):
        pltpu.make_async_copy(
            rows, out_hbm.at[pl.ds(base + c * CH, CH)], osem).wait()

    def loss(c, rows, acc):
        for g in range(CH // LANES):
            off = c * CH + g * LANES
            tcol = tgt_v[pl.ds(off, LANES)]
            ii = idx_v[pl.ds(off, LANES)]
            rowid = iota + g * LANES
            tval = plsc.load_gather(rows, [rowid, tcol])
            lval = plsc.load_gather(lse_v, [ii])
            acc = acc + (lval - tval)
        return acc

    gather_start(0, rows0, gsem0)

    def pair(j, acc):
        c0 = 2 * j
        c1 = c0 + 1
        gather_wait(c0, rows0, gsem0)
        out_start(c0, rows0, osem0)
        @pl.when(j > 0)
        def _():
            out_wait(c1 - 2, rows1, osem1)
        gather_start(c1, rows1, gsem1)
        acc = loss(c0, rows0, acc)
        gather_wait(c1, rows1, gsem1)
        out_start(c1, rows1, osem1)
        @pl.when(j < NPAIR - 1)
        def _():
            out_wait(c0, rows0, osem0)
            gather_start(c0 + 2, rows0, gsem0)
        acc = loss(c1, rows1, acc)
        return acc

    acc = lax.fori_loop(0, NPAIR, pair, jnp.zeros((LANES,), jnp.float32))
    out_wait(NCH - 2, rows0, osem0)
    out_wait(NCH - 1, rows1, osem1)
    acc_v[...] = acc
    pltpu.sync_copy(acc_v, psum_hbm.at[wid])


def kernel(idx, targets, table):
    idx_flat = idx.reshape(-1).astype(jnp.int32)
    tgt_flat = targets.reshape(-1).astype(jnp.int32)

    lse = pl.pallas_call(
        _lse_body,
        out_shape=jax.ShapeDtypeStruct((VOCAB, 1), jnp.float32),
    )(table)
    lse = lse.reshape(VOCAB)

    sc = functools.partial(
        pl.kernel,
        mesh=plsc.VectorSubcoreMesh(core_axis_name="c", subcore_axis_name="s"),
        out_type=[
            jax.ShapeDtypeStruct((N, VOCAB), jnp.float32),
            jax.ShapeDtypeStruct((NW, LANES), jnp.float32),
        ],
        scratch_types=[
            pltpu.VMEM((PER_W,), jnp.int32),
            pltpu.VMEM((PER_W,), jnp.int32),
            pltpu.VMEM((VOCAB,), jnp.float32),
            pltpu.VMEM((CH, VOCAB), jnp.float32),
            pltpu.VMEM((CH, VOCAB), jnp.float32),
            pltpu.VMEM((LANES,), jnp.float32),
            pltpu.SemaphoreType.DMA,
            pltpu.SemaphoreType.DMA,
            pltpu.SemaphoreType.DMA,
            pltpu.SemaphoreType.DMA,
        ],
        compiler_params=pltpu.CompilerParams(
            use_tc_tiling_on_sc=False, needs_layout_passes=False),
    )(_sc_gather)
    logits2, psums = sc(idx_flat, tgt_flat, table, lse)

    fin = pl.pallas_call(
        _fin_body,
        out_shape=jax.ShapeDtypeStruct((1, 1), jnp.float32),
    )(psums)
    loss = fin[0, 0]
    return (logits2, loss)
